# baseline (speedup 1.0000x reference)
import jax
import jax.numpy as jnp
from jax.experimental import pallas as pl
from jax.experimental.pallas import tpu as pltpu

MAX_M = 16384
HIDDEN = 2048
BLOCK = 1024
NB = MAX_M // BLOCK


def _copy_body(split_ref, in_ref, passthru_ref, out_ref, va, vb, sems, sema, semb):
    m = split_ref[0]
    for i in range(NB):
        lo = i * BLOCK
        hi = lo + BLOCK
        blk = pl.ds(lo, BLOCK)

        @pl.when(m >= hi)
        def _():
            pltpu.make_async_copy(in_ref.at[blk], out_ref.at[blk], sems.at[i]).start()

        @pl.when(m <= lo)
        def _():
            pltpu.make_async_copy(passthru_ref.at[blk], out_ref.at[blk], sems.at[i]).start()

    for i in range(NB):
        lo = i * BLOCK
        hi = lo + BLOCK
        blk = pl.ds(lo, BLOCK)

        @pl.when(jnp.logical_or(m >= hi, m <= lo))
        def _():
            pltpu.make_async_copy(in_ref.at[blk], out_ref.at[blk], sems.at[i]).wait()

        @pl.when(jnp.logical_and(m > lo, m < hi))
        def _():
            pltpu.make_async_copy(in_ref.at[blk], va, sema).start()
            pltpu.make_async_copy(passthru_ref.at[blk], vb, semb).start()
            pltpu.make_async_copy(in_ref.at[blk], va, sema).wait()
            pltpu.make_async_copy(passthru_ref.at[blk], vb, semb).wait()
            rows = jax.lax.broadcasted_iota(jnp.int32, (BLOCK, HIDDEN), 0) + lo
            va[...] = jnp.where(rows < m, va[...], vb[...])
            pltpu.make_async_copy(va, out_ref.at[blk], sema).start()
            pltpu.make_async_copy(va, out_ref.at[blk], sema).wait()


def kernel(input, output, input_splits, output_splits, num_sm):
    del input_splits, num_sm
    return pl.pallas_call(
        _copy_body,
        out_shape=jax.ShapeDtypeStruct((MAX_M, HIDDEN), jnp.float32),
        in_specs=[
            pl.BlockSpec(memory_space=pltpu.SMEM),
            pl.BlockSpec(memory_space=pltpu.MemorySpace.HBM),
            pl.BlockSpec(memory_space=pltpu.MemorySpace.HBM),
        ],
        out_specs=pl.BlockSpec(memory_space=pltpu.MemorySpace.HBM),
        scratch_shapes=[
            pltpu.VMEM((BLOCK, HIDDEN), jnp.float32),
            pltpu.VMEM((BLOCK, HIDDEN), jnp.float32),
            pltpu.SemaphoreType.DMA((NB,)),
            pltpu.SemaphoreType.DMA,
            pltpu.SemaphoreType.DMA,
        ],
    )(output_splits, input, output)


# speedup vs baseline: 39.4059x; 39.4059x over previous
import jax
import jax.numpy as jnp
from jax.experimental import pallas as pl
from jax.experimental.pallas import tpu as pltpu

MAX_M = 16384
HIDDEN = 2048
BLOCK = 1024
NB = MAX_M // BLOCK


def _copy_body(split_ref, in_ref, passthru_ref, out_ref, vb, sem_r, semb):
    i = pl.program_id(0)
    m = split_ref[0]
    lo = i * BLOCK
    hi = lo + BLOCK
    blk = pl.ds(lo, BLOCK)

    @pl.when(m >= hi)
    def _():
        pltpu.make_async_copy(in_ref.at[blk], out_ref, sem_r).start()
        pltpu.make_async_copy(in_ref.at[blk], out_ref, sem_r).wait()

    @pl.when(m <= lo)
    def _():
        pltpu.make_async_copy(passthru_ref.at[blk], out_ref, sem_r).start()
        pltpu.make_async_copy(passthru_ref.at[blk], out_ref, sem_r).wait()

    @pl.when(jnp.logical_and(m > lo, m < hi))
    def _():
        pltpu.make_async_copy(in_ref.at[blk], out_ref, sem_r).start()
        pltpu.make_async_copy(passthru_ref.at[blk], vb, semb).start()
        pltpu.make_async_copy(in_ref.at[blk], out_ref, sem_r).wait()
        pltpu.make_async_copy(passthru_ref.at[blk], vb, semb).wait()
        rows = jax.lax.broadcasted_iota(jnp.int32, (BLOCK, HIDDEN), 0) + lo
        out_ref[...] = jnp.where(rows < m, out_ref[...], vb[...])


def kernel(input, output, input_splits, output_splits, num_sm):
    del input_splits, num_sm
    return pl.pallas_call(
        _copy_body,
        grid=(NB,),
        out_shape=jax.ShapeDtypeStruct((MAX_M, HIDDEN), jnp.float32),
        in_specs=[
            pl.BlockSpec(memory_space=pltpu.SMEM),
            pl.BlockSpec(memory_space=pltpu.MemorySpace.HBM),
            pl.BlockSpec(memory_space=pltpu.MemorySpace.HBM),
        ],
        out_specs=pl.BlockSpec((BLOCK, HIDDEN), lambda i: (i, 0)),
        scratch_shapes=[
            pltpu.VMEM((BLOCK, HIDDEN), jnp.float32),
            pltpu.SemaphoreType.DMA,
            pltpu.SemaphoreType.DMA,
        ],
    )(output_splits, input, output)


# speedup vs baseline: 48.8685x; 1.2401x over previous
import jax
import jax.numpy as jnp
from jax.experimental import pallas as pl
from jax.experimental.pallas import tpu as pltpu

MAX_M = 16384
HIDDEN = 2048
BLOCK = 1024
NB = MAX_M // BLOCK
DEPTH = 4


def _copy_body(split_ref, in_ref, passthru_ref, out_ref, bufs, vb, sem_r, sem_w, semb):
    m = split_ref[0]

    def straddle(i):
        return jnp.logical_and(m > i * BLOCK, m < i * BLOCK + BLOCK)

    def start_read(i):
        lo = i * BLOCK
        blk = pl.ds(lo, BLOCK)
        buf = bufs.at[i % DEPTH]
        sem = sem_r.at[i % DEPTH]

        @pl.when(m >= lo + BLOCK)
        def _():
            pltpu.make_async_copy(in_ref.at[blk], buf, sem).start()

        @pl.when(m <= lo)
        def _():
            pltpu.make_async_copy(passthru_ref.at[blk], buf, sem).start()

        @pl.when(straddle(i))
        def _():
            pltpu.make_async_copy(in_ref.at[blk], buf, sem).start()
            pltpu.make_async_copy(passthru_ref.at[blk], vb, semb).start()

    def wait_read(i):
        lo = i * BLOCK
        blk = pl.ds(lo, BLOCK)
        buf = bufs.at[i % DEPTH]
        sem = sem_r.at[i % DEPTH]
        pltpu.make_async_copy(in_ref.at[blk], buf, sem).wait()

        @pl.when(straddle(i))
        def _():
            pltpu.make_async_copy(passthru_ref.at[blk], vb, semb).wait()
            rows = jax.lax.broadcasted_iota(jnp.int32, (BLOCK, HIDDEN), 0) + lo
            buf[...] = jnp.where(rows < m, buf[...], vb[...])

    def start_write(i):
        blk = pl.ds(i * BLOCK, BLOCK)
        pltpu.make_async_copy(
            bufs.at[i % DEPTH], out_ref.at[blk], sem_w.at[i % DEPTH]
        ).start()

    def wait_write(i):
        blk = pl.ds(i * BLOCK, BLOCK)
        pltpu.make_async_copy(
            bufs.at[i % DEPTH], out_ref.at[blk], sem_w.at[i % DEPTH]
        ).wait()

    for i in range(NB):
        if i >= DEPTH:
            wait_write(i - DEPTH)
        start_read(i)
        if i >= 1:
            wait_read(i - 1)
            start_write(i - 1)
    wait_read(NB - 1)
    start_write(NB - 1)
    for i in range(max(0, NB - DEPTH), NB):
        wait_write(i)


def kernel(input, output, input_splits, output_splits, num_sm):
    del input_splits, num_sm
    return pl.pallas_call(
        _copy_body,
        out_shape=jax.ShapeDtypeStruct((MAX_M, HIDDEN), jnp.float32),
        in_specs=[
            pl.BlockSpec(memory_space=pltpu.SMEM),
            pl.BlockSpec(memory_space=pltpu.MemorySpace.HBM),
            pl.BlockSpec(memory_space=pltpu.MemorySpace.HBM),
        ],
        out_specs=pl.BlockSpec(memory_space=pltpu.MemorySpace.HBM),
        scratch_shapes=[
            pltpu.VMEM((DEPTH, BLOCK, HIDDEN), jnp.float32),
            pltpu.VMEM((BLOCK, HIDDEN), jnp.float32),
            pltpu.SemaphoreType.DMA((DEPTH,)),
            pltpu.SemaphoreType.DMA((DEPTH,)),
            pltpu.SemaphoreType.DMA,
        ],
    )(output_splits, input, output)
